# baseline (speedup 1.0000x reference)
import jax
import jax.numpy as jnp
from jax import lax
from jax.experimental import pallas as pl
from jax.experimental.pallas import tpu as pltpu
from jax.experimental.pallas import tpu_sc as plsc

N = 10000
E = 320000
D = 128
ALPHA = 0.2
RATE = 0.5

NS = 16
CHUNK = 80
RING = 2
NCHUNK = 252
EPT = NCHUNK * CHUNK
E_PAD = NS * EPT
RPT = 632
N_PAD = NS * RPT


def _spmm_body(x0, src0, dst0, x1, src1, dst1, zeros, out0, out1, *scr):
    si = scr[0:RING]
    di = scr[RING:2 * RING]
    rows = scr[2 * RING:3 * RING]
    acc = scr[3 * RING]
    b = 3 * RING + 1
    sr = scr[b:b + RING]
    sd = scr[b + RING:b + 2 * RING]
    sg = scr[b + 2 * RING:b + 3 * RING]
    ss = scr[b + 3 * RING:b + 4 * RING]
    cid = lax.axis_index("c")
    sid = lax.axis_index("s")

    def run(x, src, dst, out):
        r0 = sid * RPT
        pltpu.sync_copy(zeros.at[pl.ds(r0, RPT)], acc.at[pl.ds(r0, RPT)])
        plsc.subcore_barrier()
        base = sid * EPT

        def idx(arr, v, buf, sem):
            off = pl.multiple_of(base + v * CHUNK, CHUNK)
            pltpu.async_copy(arr.at[pl.ds(off, CHUNK)], buf, sem)

        for k in range(RING):
            idx(src, k, si[k], sr[k])
        for k in range(2):
            idx(dst, k, di[k], sd[k])
        for k in range(2):
            pltpu.make_async_copy(src.at[pl.ds(base, CHUNK)], si[k], sr[k]).wait()
            pltpu.async_copy(x.at[si[k]], rows[k], sg[k])

        def visit(v, k):
            b2 = (k + 2) % RING
            pltpu.make_async_copy(x.at[si[k]], rows[k], sg[k]).wait()
            pltpu.make_async_copy(dst.at[pl.ds(base, CHUNK)], di[k], sd[k]).wait()
            pltpu.sync_copy(rows[k], acc.at[di[k]], add=True)

            @pl.when(v + RING < NCHUNK)
            def _():
                idx(src, v + RING, si[k], sr[k])

            @pl.when(v + 2 < NCHUNK)
            def _():
                idx(dst, v + 2, di[b2], sd[b2])
                pltpu.make_async_copy(src.at[pl.ds(base, CHUNK)], si[b2], sr[b2]).wait()
                pltpu.async_copy(x.at[si[b2]], rows[b2], sg[b2])

        def block(j, carry):
            for k in range(RING):
                visit(RING * j + k, k)
            return carry

        lax.fori_loop(0, NCHUNK // RING, block, 0)
        plsc.subcore_barrier()
        pltpu.sync_copy(acc.at[pl.ds(r0, RPT)], out.at[pl.ds(r0, RPT)])

    @pl.when(cid == 0)
    def _():
        run(x0, src0, dst0, out0)

    @pl.when(cid == 1)
    def _():
        run(x1, src1, dst1, out1)


def _spmm_pair(x0, src0, dst0, x1, src1, dst1, zeros):
    f = pl.kernel(
        _spmm_body,
        out_type=(jax.ShapeDtypeStruct((N_PAD, D), jnp.float32),
                  jax.ShapeDtypeStruct((N_PAD, D), jnp.float32)),
        mesh=plsc.VectorSubcoreMesh(core_axis_name="c", subcore_axis_name="s"),
        scratch_types=(
            [pltpu.VMEM((CHUNK,), jnp.int32)] * (2 * RING)
            + [pltpu.VMEM((CHUNK, D), jnp.float32)] * RING
            + [pltpu.VMEM_SHARED((N_PAD, D), jnp.float32)]
            + [pltpu.SemaphoreType.DMA] * (4 * RING)
        ),
    )
    return f(x0, src0, dst0, x1, src1, dst1, zeros)


def _leaky(x):
    return jnp.where(x >= 0, x, ALPHA * x)


BT = 2000
BT_H = 1264


def _hidden_body(p0, p1, w0, w1, h0, h1):
    h0[...] = _leaky(jnp.dot(p0[...], w0[...], preferred_element_type=jnp.float32))
    h1[...] = _leaky(jnp.dot(p1[...], w1[...], preferred_element_type=jnp.float32))


def _hidden(p10, p11, W1_0, W1_1):
    row = pl.BlockSpec((BT_H, D), lambda i: (i, 0))
    wsp = pl.BlockSpec((D, D), lambda i: (0, 0))
    return pl.pallas_call(
        _hidden_body,
        grid=(N_PAD // BT_H,),
        in_specs=[row, row, wsp, wsp],
        out_specs=[row, row],
        out_shape=(jax.ShapeDtypeStruct((N_PAD, D), jnp.float32),
                   jax.ShapeDtypeStruct((N_PAD, D), jnp.float32)),
    )(p10, p11, W1_0, W1_1)


def _final_body(p20, uf0, p21, uf1,
                w2m0, w2s0, umt0, umb0, ust0, usb0, mb0, sb0,
                w2m1, w2s1, umt1, umb1, ust1, usb1, mb1, sb1,
                vm, vs):
    def dom(p2, uf, w2m, w2s, umt, umb, ust, usb, mb, sb):
        mh = _leaky(jnp.dot(p2[...], w2m[...], preferred_element_type=jnp.float32))
        lh = _leaky(jnp.dot(p2[...], w2s[...], preferred_element_type=jnp.float32))
        mean = (jnp.dot(mh, umt[...], preferred_element_type=jnp.float32)
                + jnp.dot(uf[...], umb[...], preferred_element_type=jnp.float32)
                + mb[...])
        logstd = (jnp.dot(lh, ust[...], preferred_element_type=jnp.float32)
                  + jnp.dot(uf[...], usb[...], preferred_element_type=jnp.float32)
                  + sb[...])
        return mean, logstd

    m0, s0 = dom(p20, uf0, w2m0, w2s0, umt0, umb0, ust0, usb0, mb0, sb0)
    m1, s1 = dom(p21, uf1, w2m1, w2s1, umt1, umb1, ust1, usb1, mb1, sb1)
    vm[...] = RATE * m0 + (1.0 - RATE) * m1
    vs[...] = RATE * s0 + (1.0 - RATE) * s1


def _final(p20, uf0, p21, uf1, weights):
    row = pl.BlockSpec((BT, D), lambda i: (i, 0))
    wsp = pl.BlockSpec((D, D), lambda i: (0, 0))
    bsp = pl.BlockSpec((1, D), lambda i: (0, 0))
    w_specs = ([wsp] * 6 + [bsp] * 2) * 2
    return pl.pallas_call(
        _final_body,
        grid=(N // BT,),
        in_specs=[row, row, row, row] + w_specs,
        out_specs=[row, row],
        out_shape=(jax.ShapeDtypeStruct((N, D), jnp.float32),
                   jax.ShapeDtypeStruct((N, D), jnp.float32)),
    )(p20, uf0, p21, uf1, *weights)


def kernel(ufea_0, ufea_1, edge_index_0, edge_index_1,
           W1_0, W2m_0, W2s_0, UmW_0, Umb_0, UsW_0, Usb_0,
           W1_1, W2m_1, W2s_1, UmW_1, Umb_1, UsW_1, Usb_1):
    pad_e = jnp.full((2, E_PAD - E), N_PAD - 1, jnp.int32)
    ei0 = jnp.concatenate([edge_index_0, pad_e], axis=1)
    ei1 = jnp.concatenate([edge_index_1, pad_e], axis=1)
    src0, dst0 = ei0[0], ei0[1]
    src1, dst1 = ei1[0], ei1[1]
    pad_x = jnp.zeros((N_PAD - N, D), jnp.float32)
    uf0p = jnp.concatenate([ufea_0, pad_x], axis=0)
    uf1p = jnp.concatenate([ufea_1, pad_x], axis=0)
    zeros = jnp.zeros((N_PAD, D), jnp.float32)

    p10, p11 = _spmm_pair(uf0p, src0, dst0, uf1p, src1, dst1, zeros)
    h0, h1 = _hidden(p10, p11, W1_0, W1_1)
    p20, p21 = _spmm_pair(h0, dst0, src0, h1, dst1, src1, zeros)

    weights = (
        W2m_0, W2s_0, UmW_0[:D], UmW_0[D:], UsW_0[:D], UsW_0[D:],
        Umb_0.reshape(1, D), Usb_0.reshape(1, D),
        W2m_1, W2s_1, UmW_1[:D], UmW_1[D:], UsW_1[:D], UsW_1[D:],
        Umb_1.reshape(1, D), Usb_1.reshape(1, D),
    )
    return _final(p20, ufea_0, p21, ufea_1, weights)


# speedup vs baseline: 1.3290x; 1.3290x over previous
import jax
import jax.numpy as jnp
from jax import lax
from jax.experimental import pallas as pl
from jax.experimental.pallas import tpu as pltpu
from jax.experimental.pallas import tpu_sc as plsc

N = 10000
E = 320000
D = 128
ALPHA = 0.2
RATE = 0.5

NS = 16
CHUNK = 80
RING = 2
NCHUNK = 252
EPT = NCHUNK * CHUNK
E_PAD = NS * EPT
RPT = 632
N_PAD = NS * RPT


def _spmm_body(x0, src0, dst0, x1, src1, dst1, zeros, out0, out1, *scr):
    si = scr[0:RING]
    di = scr[RING:2 * RING]
    rows = scr[2 * RING:3 * RING]
    acc = scr[3 * RING]
    b = 3 * RING + 1
    sr = scr[b:b + RING]
    sd = scr[b + RING:b + 2 * RING]
    sg = scr[b + 2 * RING:b + 3 * RING]
    ss = scr[b + 3 * RING:b + 4 * RING]
    cid = lax.axis_index("c")
    sid = lax.axis_index("s")

    def run(x, src, dst, out):
        r0 = sid * RPT
        pltpu.sync_copy(zeros.at[pl.ds(r0, RPT)], acc.at[pl.ds(r0, RPT)])
        plsc.subcore_barrier()
        base = sid * EPT

        def idx(arr, v, buf, sem):
            off = pl.multiple_of(base + v * CHUNK, CHUNK)
            pltpu.async_copy(arr.at[pl.ds(off, CHUNK)], buf, sem)

        for k in range(RING):
            idx(src, k, si[k], sr[k])
        for k in range(2):
            idx(dst, k, di[k], sd[k])
        for k in range(2):
            pltpu.make_async_copy(src.at[pl.ds(base, CHUNK)], si[k], sr[k]).wait()
            pltpu.async_copy(x.at[si[k]], rows[k], sg[k])

        def visit(v, k):
            b2 = (k + 2) % RING
            pltpu.make_async_copy(x.at[si[k]], rows[k], sg[k]).wait()
            pltpu.make_async_copy(dst.at[pl.ds(base, CHUNK)], di[k], sd[k]).wait()
            pltpu.sync_copy(rows[k], acc.at[di[k]], add=True)

            @pl.when(v + RING < NCHUNK)
            def _():
                idx(src, v + RING, si[k], sr[k])

            @pl.when(v + 2 < NCHUNK)
            def _():
                idx(dst, v + 2, di[b2], sd[b2])
                pltpu.make_async_copy(src.at[pl.ds(base, CHUNK)], si[b2], sr[b2]).wait()
                pltpu.async_copy(x.at[si[b2]], rows[b2], sg[b2])

        def block(j, carry):
            for k in range(RING):
                visit(RING * j + k, k)
            return carry

        lax.fori_loop(0, NCHUNK // RING, block, 0)
        plsc.subcore_barrier()
        pltpu.sync_copy(acc.at[pl.ds(r0, RPT)], out.at[pl.ds(r0, RPT)])

    @pl.when(cid == 0)
    def _():
        run(x0, src0, dst0, out0)

    @pl.when(cid == 1)
    def _():
        run(x1, src1, dst1, out1)


def _spmm_pair(x0, src0, dst0, x1, src1, dst1, zeros):
    f = pl.kernel(
        _spmm_body,
        out_type=(jax.ShapeDtypeStruct((N_PAD, D), jnp.float32),
                  jax.ShapeDtypeStruct((N_PAD, D), jnp.float32)),
        mesh=plsc.VectorSubcoreMesh(core_axis_name="c", subcore_axis_name="s"),
        scratch_types=(
            [pltpu.VMEM((CHUNK,), jnp.int32)] * (2 * RING)
            + [pltpu.VMEM((CHUNK, D), jnp.float32)] * RING
            + [pltpu.VMEM_SHARED((N_PAD, D), jnp.float32)]
            + [pltpu.SemaphoreType.DMA] * (4 * RING)
        ),
    )
    return f(x0, src0, dst0, x1, src1, dst1, zeros)


def _leaky(x):
    return jnp.where(x >= 0, x, ALPHA * x)


BT = 2000
BT_H = 1264


def _hidden_body(p0, p1, w0, w1, h0, h1):
    h0[...] = _leaky(jnp.dot(p0[...], w0[...], preferred_element_type=jnp.float32))
    h1[...] = _leaky(jnp.dot(p1[...], w1[...], preferred_element_type=jnp.float32))


def _hidden(p10, p11, W1_0, W1_1):
    row = pl.BlockSpec((BT_H, D), lambda i: (i, 0))
    wsp = pl.BlockSpec((D, D), lambda i: (0, 0))
    return pl.pallas_call(
        _hidden_body,
        grid=(N_PAD // BT_H,),
        in_specs=[row, row, wsp, wsp],
        out_specs=[row, row],
        out_shape=(jax.ShapeDtypeStruct((N_PAD, D), jnp.float32),
                   jax.ShapeDtypeStruct((N_PAD, D), jnp.float32)),
    )(p10, p11, W1_0, W1_1)


def _final_body(p20, uf0, p21, uf1,
                w2m0, w2s0, umt0, umb0, ust0, usb0, mb0, sb0,
                w2m1, w2s1, umt1, umb1, ust1, usb1, mb1, sb1,
                vm, vs):
    def dom(p2, uf, w2m, w2s, umt, umb, ust, usb, mb, sb):
        mh = _leaky(jnp.dot(p2[...], w2m[...], preferred_element_type=jnp.float32))
        lh = _leaky(jnp.dot(p2[...], w2s[...], preferred_element_type=jnp.float32))
        mean = (jnp.dot(mh, umt[...], preferred_element_type=jnp.float32)
                + jnp.dot(uf[...], umb[...], preferred_element_type=jnp.float32)
                + mb[...])
        logstd = (jnp.dot(lh, ust[...], preferred_element_type=jnp.float32)
                  + jnp.dot(uf[...], usb[...], preferred_element_type=jnp.float32)
                  + sb[...])
        return mean, logstd

    m0, s0 = dom(p20, uf0, w2m0, w2s0, umt0, umb0, ust0, usb0, mb0, sb0)
    m1, s1 = dom(p21, uf1, w2m1, w2s1, umt1, umb1, ust1, usb1, mb1, sb1)
    vm[...] = RATE * m0 + (1.0 - RATE) * m1
    vs[...] = RATE * s0 + (1.0 - RATE) * s1


def _final(p20, uf0, p21, uf1, weights):
    row = pl.BlockSpec((BT, D), lambda i: (i, 0))
    wsp = pl.BlockSpec((D, D), lambda i: (0, 0))
    bsp = pl.BlockSpec((1, D), lambda i: (0, 0))
    w_specs = ([wsp] * 6 + [bsp] * 2) * 2
    return pl.pallas_call(
        _final_body,
        grid=(N // BT,),
        in_specs=[row, row, row, row] + w_specs,
        out_specs=[row, row],
        out_shape=(jax.ShapeDtypeStruct((N, D), jnp.float32),
                   jax.ShapeDtypeStruct((N, D), jnp.float32)),
    )(p20, uf0, p21, uf1, *weights)


def kernel(ufea_0, ufea_1, edge_index_0, edge_index_1,
           W1_0, W2m_0, W2s_0, UmW_0, Umb_0, UsW_0, Usb_0,
           W1_1, W2m_1, W2s_1, UmW_1, Umb_1, UsW_1, Usb_1):
    npad = E_PAD - E
    pad_src = N + (jnp.arange(npad, dtype=jnp.int32) % (N_PAD - N))
    pad_dst = jnp.arange(npad, dtype=jnp.int32) % N
    pad_e = jnp.stack([pad_src, pad_dst])
    ei0 = jnp.concatenate([edge_index_0, pad_e], axis=1)
    ei1 = jnp.concatenate([edge_index_1, pad_e], axis=1)
    src0, dst0 = ei0[0], ei0[1]
    src1, dst1 = ei1[0], ei1[1]
    pad_x = jnp.zeros((N_PAD - N, D), jnp.float32)
    uf0p = jnp.concatenate([ufea_0, pad_x], axis=0)
    uf1p = jnp.concatenate([ufea_1, pad_x], axis=0)
    zeros = jnp.zeros((N_PAD, D), jnp.float32)

    p10, p11 = _spmm_pair(uf0p, src0, dst0, uf1p, src1, dst1, zeros)
    h0, h1 = _hidden(p10, p11, W1_0, W1_1)
    p20, p21 = _spmm_pair(h0, dst0, src0, h1, dst1, src1, zeros)

    weights = (
        W2m_0, W2s_0, UmW_0[:D], UmW_0[D:], UsW_0[:D], UsW_0[D:],
        Umb_0.reshape(1, D), Usb_0.reshape(1, D),
        W2m_1, W2s_1, UmW_1[:D], UmW_1[D:], UsW_1[:D], UsW_1[D:],
        Umb_1.reshape(1, D), Usb_1.reshape(1, D),
    )
    return _final(p20, ufea_0, p21, ufea_1, weights)


# speedup vs baseline: 1.6123x; 1.2131x over previous
import jax
import jax.numpy as jnp
from jax import lax
from jax.experimental import pallas as pl
from jax.experimental.pallas import tpu as pltpu
from jax.experimental.pallas import tpu_sc as plsc

N = 10000
E = 320000
D = 128
ALPHA = 0.2
RATE = 0.5

NS = 16
CHUNK = 80
RING = 3
NCHUNK = 252
EPT = NCHUNK * CHUNK
E_PAD = NS * EPT
RPT = 632
N_PAD = NS * RPT


def _spmm_body(x0, src0, dst0, x1, src1, dst1, zeros, out0, out1, *scr):
    si = scr[0:RING]
    di = scr[RING:2 * RING]
    rows = scr[2 * RING:3 * RING]
    acc = scr[3 * RING]
    b = 3 * RING + 1
    sr = scr[b:b + RING]
    sd = scr[b + RING:b + 2 * RING]
    sg = scr[b + 2 * RING:b + 3 * RING]
    ss = scr[b + 3 * RING:b + 4 * RING]
    cid = lax.axis_index("c")
    sid = lax.axis_index("s")

    def run(x, src, dst, out):
        r0 = sid * RPT
        pltpu.sync_copy(zeros.at[pl.ds(r0, RPT)], acc.at[pl.ds(r0, RPT)])
        plsc.subcore_barrier()
        base = sid * EPT

        def idx(arr, v, buf, sem):
            off = pl.multiple_of(base + v * CHUNK, CHUNK)
            pltpu.async_copy(arr.at[pl.ds(off, CHUNK)], buf, sem)

        for k in range(RING):
            idx(src, k, si[k], sr[k])
        for k in range(2):
            idx(dst, k, di[k], sd[k])
        for k in range(2):
            pltpu.make_async_copy(src.at[pl.ds(base, CHUNK)], si[k], sr[k]).wait()
            pltpu.async_copy(x.at[si[k]], rows[k], sg[k])

        def visit(v, k):
            b2 = (k + 2) % RING
            pltpu.make_async_copy(x.at[si[k]], rows[k], sg[k]).wait()
            pltpu.make_async_copy(dst.at[pl.ds(base, CHUNK)], di[k], sd[k]).wait()
            pltpu.sync_copy(rows[k], acc.at[di[k]], add=True)

            @pl.when(v + RING < NCHUNK)
            def _():
                idx(src, v + RING, si[k], sr[k])

            @pl.when(v + 2 < NCHUNK)
            def _():
                idx(dst, v + 2, di[b2], sd[b2])
                pltpu.make_async_copy(src.at[pl.ds(base, CHUNK)], si[b2], sr[b2]).wait()
                pltpu.async_copy(x.at[si[b2]], rows[b2], sg[b2])

        def block(j, carry):
            for k in range(RING):
                visit(RING * j + k, k)
            return carry

        lax.fori_loop(0, NCHUNK // RING, block, 0)
        plsc.subcore_barrier()
        pltpu.sync_copy(acc.at[pl.ds(r0, RPT)], out.at[pl.ds(r0, RPT)])

    @pl.when(cid == 0)
    def _():
        run(x0, src0, dst0, out0)

    @pl.when(cid == 1)
    def _():
        run(x1, src1, dst1, out1)


def _spmm_pair(x0, src0, dst0, x1, src1, dst1, zeros):
    f = pl.kernel(
        _spmm_body,
        out_type=(jax.ShapeDtypeStruct((N_PAD, D), jnp.float32),
                  jax.ShapeDtypeStruct((N_PAD, D), jnp.float32)),
        mesh=plsc.VectorSubcoreMesh(core_axis_name="c", subcore_axis_name="s"),
        scratch_types=(
            [pltpu.VMEM((CHUNK,), jnp.int32)] * (2 * RING)
            + [pltpu.VMEM((CHUNK, D), jnp.float32)] * RING
            + [pltpu.VMEM_SHARED((N_PAD, D), jnp.float32)]
            + [pltpu.SemaphoreType.DMA] * (4 * RING)
        ),
    )
    return f(x0, src0, dst0, x1, src1, dst1, zeros)


def _leaky(x):
    return jnp.where(x >= 0, x, ALPHA * x)


BT = 2000
BT_H = 1264


def _hidden_body(p0, p1, w0, w1, h0, h1):
    h0[...] = _leaky(jnp.dot(p0[...], w0[...], preferred_element_type=jnp.float32))
    h1[...] = _leaky(jnp.dot(p1[...], w1[...], preferred_element_type=jnp.float32))


def _hidden(p10, p11, W1_0, W1_1):
    row = pl.BlockSpec((BT_H, D), lambda i: (i, 0))
    wsp = pl.BlockSpec((D, D), lambda i: (0, 0))
    return pl.pallas_call(
        _hidden_body,
        grid=(N_PAD // BT_H,),
        in_specs=[row, row, wsp, wsp],
        out_specs=[row, row],
        out_shape=(jax.ShapeDtypeStruct((N_PAD, D), jnp.float32),
                   jax.ShapeDtypeStruct((N_PAD, D), jnp.float32)),
    )(p10, p11, W1_0, W1_1)


def _final_body(p20, uf0, p21, uf1,
                w2m0, w2s0, umt0, umb0, ust0, usb0, mb0, sb0,
                w2m1, w2s1, umt1, umb1, ust1, usb1, mb1, sb1,
                vm, vs):
    def dom(p2, uf, w2m, w2s, umt, umb, ust, usb, mb, sb):
        mh = _leaky(jnp.dot(p2[...], w2m[...], preferred_element_type=jnp.float32))
        lh = _leaky(jnp.dot(p2[...], w2s[...], preferred_element_type=jnp.float32))
        mean = (jnp.dot(mh, umt[...], preferred_element_type=jnp.float32)
                + jnp.dot(uf[...], umb[...], preferred_element_type=jnp.float32)
                + mb[...])
        logstd = (jnp.dot(lh, ust[...], preferred_element_type=jnp.float32)
                  + jnp.dot(uf[...], usb[...], preferred_element_type=jnp.float32)
                  + sb[...])
        return mean, logstd

    m0, s0 = dom(p20, uf0, w2m0, w2s0, umt0, umb0, ust0, usb0, mb0, sb0)
    m1, s1 = dom(p21, uf1, w2m1, w2s1, umt1, umb1, ust1, usb1, mb1, sb1)
    vm[...] = RATE * m0 + (1.0 - RATE) * m1
    vs[...] = RATE * s0 + (1.0 - RATE) * s1


def _final(p20, uf0, p21, uf1, weights):
    row = pl.BlockSpec((BT, D), lambda i: (i, 0))
    wsp = pl.BlockSpec((D, D), lambda i: (0, 0))
    bsp = pl.BlockSpec((1, D), lambda i: (0, 0))
    w_specs = ([wsp] * 6 + [bsp] * 2) * 2
    return pl.pallas_call(
        _final_body,
        grid=(N // BT,),
        in_specs=[row, row, row, row] + w_specs,
        out_specs=[row, row],
        out_shape=(jax.ShapeDtypeStruct((N, D), jnp.float32),
                   jax.ShapeDtypeStruct((N, D), jnp.float32)),
    )(p20, uf0, p21, uf1, *weights)


def kernel(ufea_0, ufea_1, edge_index_0, edge_index_1,
           W1_0, W2m_0, W2s_0, UmW_0, Umb_0, UsW_0, Usb_0,
           W1_1, W2m_1, W2s_1, UmW_1, Umb_1, UsW_1, Usb_1):
    npad = E_PAD - E
    pad_src = N + (jnp.arange(npad, dtype=jnp.int32) % (N_PAD - N))
    pad_dst = jnp.arange(npad, dtype=jnp.int32) % N
    pad_e = jnp.stack([pad_src, pad_dst])
    ei0 = jnp.concatenate([edge_index_0, pad_e], axis=1)
    ei1 = jnp.concatenate([edge_index_1, pad_e], axis=1)
    src0, dst0 = ei0[0], ei0[1]
    src1, dst1 = ei1[0], ei1[1]
    pad_x = jnp.zeros((N_PAD - N, D), jnp.float32)
    uf0p = jnp.concatenate([ufea_0, pad_x], axis=0)
    uf1p = jnp.concatenate([ufea_1, pad_x], axis=0)
    zeros = jnp.zeros((N_PAD, D), jnp.float32)

    p10, p11 = _spmm_pair(uf0p, src0, dst0, uf1p, src1, dst1, zeros)
    h0, h1 = _hidden(p10, p11, W1_0, W1_1)
    p20, p21 = _spmm_pair(h0, dst0, src0, h1, dst1, src1, zeros)

    weights = (
        W2m_0, W2s_0, UmW_0[:D], UmW_0[D:], UsW_0[:D], UsW_0[D:],
        Umb_0.reshape(1, D), Usb_0.reshape(1, D),
        W2m_1, W2s_1, UmW_1[:D], UmW_1[D:], UsW_1[:D], UsW_1[D:],
        Umb_1.reshape(1, D), Usb_1.reshape(1, D),
    )
    return _final(p20, ufea_0, p21, ufea_1, weights)


# speedup vs baseline: 1.7796x; 1.1038x over previous
import jax
import jax.numpy as jnp
from jax import lax
from jax.experimental import pallas as pl
from jax.experimental.pallas import tpu as pltpu
from jax.experimental.pallas import tpu_sc as plsc

N = 10000
E = 320000
D = 128
ALPHA = 0.2
RATE = 0.5

NS = 16
CHUNK = 128
RING = 3
NCHUNK = 159
EPT = NCHUNK * CHUNK
E_PAD = NS * EPT
RPT = 632
N_PAD = NS * RPT


def _spmm_body(x0, src0, dst0, x1, src1, dst1, zeros, out0, out1, *scr):
    si = scr[0:RING]
    di = scr[RING:2 * RING]
    rows = scr[2 * RING:3 * RING]
    acc = scr[3 * RING]
    b = 3 * RING + 1
    sr = scr[b:b + RING]
    sd = scr[b + RING:b + 2 * RING]
    sg = scr[b + 2 * RING:b + 3 * RING]
    ss = scr[b + 3 * RING:b + 4 * RING]
    cid = lax.axis_index("c")
    sid = lax.axis_index("s")

    def run(x, src, dst, out):
        r0 = sid * RPT
        pltpu.sync_copy(zeros.at[pl.ds(r0, RPT)], acc.at[pl.ds(r0, RPT)])
        plsc.subcore_barrier()
        base = sid * EPT

        def idx(arr, v, buf, sem):
            off = pl.multiple_of(base + v * CHUNK, CHUNK)
            pltpu.async_copy(arr.at[pl.ds(off, CHUNK)], buf, sem)

        for k in range(RING):
            idx(src, k, si[k], sr[k])
        for k in range(2):
            idx(dst, k, di[k], sd[k])
        for k in range(2):
            pltpu.make_async_copy(src.at[pl.ds(base, CHUNK)], si[k], sr[k]).wait()
            pltpu.async_copy(x.at[si[k]], rows[k], sg[k])

        def visit(v, k):
            b2 = (k + 2) % RING
            pltpu.make_async_copy(x.at[si[k]], rows[k], sg[k]).wait()
            pltpu.make_async_copy(dst.at[pl.ds(base, CHUNK)], di[k], sd[k]).wait()
            pltpu.sync_copy(rows[k], acc.at[di[k]], add=True)

            @pl.when(v + RING < NCHUNK)
            def _():
                idx(src, v + RING, si[k], sr[k])

            @pl.when(v + 2 < NCHUNK)
            def _():
                idx(dst, v + 2, di[b2], sd[b2])
                pltpu.make_async_copy(src.at[pl.ds(base, CHUNK)], si[b2], sr[b2]).wait()
                pltpu.async_copy(x.at[si[b2]], rows[b2], sg[b2])

        def block(j, carry):
            for k in range(RING):
                visit(RING * j + k, k)
            return carry

        lax.fori_loop(0, NCHUNK // RING, block, 0)
        plsc.subcore_barrier()
        pltpu.sync_copy(acc.at[pl.ds(r0, RPT)], out.at[pl.ds(r0, RPT)])

    @pl.when(cid == 0)
    def _():
        run(x0, src0, dst0, out0)

    @pl.when(cid == 1)
    def _():
        run(x1, src1, dst1, out1)


def _spmm_pair(x0, src0, dst0, x1, src1, dst1, zeros):
    f = pl.kernel(
        _spmm_body,
        out_type=(jax.ShapeDtypeStruct((N_PAD, D), jnp.float32),
                  jax.ShapeDtypeStruct((N_PAD, D), jnp.float32)),
        mesh=plsc.VectorSubcoreMesh(core_axis_name="c", subcore_axis_name="s"),
        scratch_types=(
            [pltpu.VMEM((CHUNK,), jnp.int32)] * (2 * RING)
            + [pltpu.VMEM((CHUNK, D), jnp.float32)] * RING
            + [pltpu.VMEM_SHARED((N_PAD, D), jnp.float32)]
            + [pltpu.SemaphoreType.DMA] * (4 * RING)
        ),
    )
    return f(x0, src0, dst0, x1, src1, dst1, zeros)


def _leaky(x):
    return jnp.where(x >= 0, x, ALPHA * x)


BT = 2000
BT_H = 1264


def _hidden_body(p0, p1, w0, w1, h0, h1):
    h0[...] = _leaky(jnp.dot(p0[...], w0[...], preferred_element_type=jnp.float32))
    h1[...] = _leaky(jnp.dot(p1[...], w1[...], preferred_element_type=jnp.float32))


def _hidden(p10, p11, W1_0, W1_1):
    row = pl.BlockSpec((BT_H, D), lambda i: (i, 0))
    wsp = pl.BlockSpec((D, D), lambda i: (0, 0))
    return pl.pallas_call(
        _hidden_body,
        grid=(N_PAD // BT_H,),
        in_specs=[row, row, wsp, wsp],
        out_specs=[row, row],
        out_shape=(jax.ShapeDtypeStruct((N_PAD, D), jnp.float32),
                   jax.ShapeDtypeStruct((N_PAD, D), jnp.float32)),
    )(p10, p11, W1_0, W1_1)


def _final_body(p20, uf0, p21, uf1,
                w2m0, w2s0, umt0, umb0, ust0, usb0, mb0, sb0,
                w2m1, w2s1, umt1, umb1, ust1, usb1, mb1, sb1,
                vm, vs):
    def dom(p2, uf, w2m, w2s, umt, umb, ust, usb, mb, sb):
        mh = _leaky(jnp.dot(p2[...], w2m[...], preferred_element_type=jnp.float32))
        lh = _leaky(jnp.dot(p2[...], w2s[...], preferred_element_type=jnp.float32))
        mean = (jnp.dot(mh, umt[...], preferred_element_type=jnp.float32)
                + jnp.dot(uf[...], umb[...], preferred_element_type=jnp.float32)
                + mb[...])
        logstd = (jnp.dot(lh, ust[...], preferred_element_type=jnp.float32)
                  + jnp.dot(uf[...], usb[...], preferred_element_type=jnp.float32)
                  + sb[...])
        return mean, logstd

    m0, s0 = dom(p20, uf0, w2m0, w2s0, umt0, umb0, ust0, usb0, mb0, sb0)
    m1, s1 = dom(p21, uf1, w2m1, w2s1, umt1, umb1, ust1, usb1, mb1, sb1)
    vm[...] = RATE * m0 + (1.0 - RATE) * m1
    vs[...] = RATE * s0 + (1.0 - RATE) * s1


def _final(p20, uf0, p21, uf1, weights):
    row = pl.BlockSpec((BT, D), lambda i: (i, 0))
    wsp = pl.BlockSpec((D, D), lambda i: (0, 0))
    bsp = pl.BlockSpec((1, D), lambda i: (0, 0))
    w_specs = ([wsp] * 6 + [bsp] * 2) * 2
    return pl.pallas_call(
        _final_body,
        grid=(N // BT,),
        in_specs=[row, row, row, row] + w_specs,
        out_specs=[row, row],
        out_shape=(jax.ShapeDtypeStruct((N, D), jnp.float32),
                   jax.ShapeDtypeStruct((N, D), jnp.float32)),
    )(p20, uf0, p21, uf1, *weights)


def kernel(ufea_0, ufea_1, edge_index_0, edge_index_1,
           W1_0, W2m_0, W2s_0, UmW_0, Umb_0, UsW_0, Usb_0,
           W1_1, W2m_1, W2s_1, UmW_1, Umb_1, UsW_1, Usb_1):
    npad = E_PAD - E
    pad_src = N + (jnp.arange(npad, dtype=jnp.int32) % (N_PAD - N))
    pad_dst = jnp.arange(npad, dtype=jnp.int32) % N
    pad_e = jnp.stack([pad_src, pad_dst])
    ei0 = jnp.concatenate([edge_index_0, pad_e], axis=1)
    ei1 = jnp.concatenate([edge_index_1, pad_e], axis=1)
    src0, dst0 = ei0[0], ei0[1]
    src1, dst1 = ei1[0], ei1[1]
    pad_x = jnp.zeros((N_PAD - N, D), jnp.float32)
    uf0p = jnp.concatenate([ufea_0, pad_x], axis=0)
    uf1p = jnp.concatenate([ufea_1, pad_x], axis=0)
    zeros = jnp.zeros((N_PAD, D), jnp.float32)

    p10, p11 = _spmm_pair(uf0p, src0, dst0, uf1p, src1, dst1, zeros)
    h0, h1 = _hidden(p10, p11, W1_0, W1_1)
    p20, p21 = _spmm_pair(h0, dst0, src0, h1, dst1, src1, zeros)

    weights = (
        W2m_0, W2s_0, UmW_0[:D], UmW_0[D:], UsW_0[:D], UsW_0[D:],
        Umb_0.reshape(1, D), Usb_0.reshape(1, D),
        W2m_1, W2s_1, UmW_1[:D], UmW_1[D:], UsW_1[:D], UsW_1[D:],
        Umb_1.reshape(1, D), Usb_1.reshape(1, D),
    )
    return _final(p20, ufea_0, p21, ufea_1, weights)


# speedup vs baseline: 1.9819x; 1.1137x over previous
import jax
import jax.numpy as jnp
from jax import lax
from jax.experimental import pallas as pl
from jax.experimental.pallas import tpu as pltpu
from jax.experimental.pallas import tpu_sc as plsc

N = 10000
E = 320000
D = 128
ALPHA = 0.2
RATE = 0.5

NS = 16
CHUNK = 96
RING = 4
NCHUNK = 212
EPT = NCHUNK * CHUNK
E_PAD = NS * EPT
RPT = 632
N_PAD = NS * RPT


def _spmm_body(x0, src0, dst0, x1, src1, dst1, zeros, out0, out1, *scr):
    si = scr[0:RING]
    di = scr[RING:2 * RING]
    rows = scr[2 * RING:3 * RING]
    acc = scr[3 * RING]
    b = 3 * RING + 1
    sr = scr[b:b + RING]
    sd = scr[b + RING:b + 2 * RING]
    sg = scr[b + 2 * RING:b + 3 * RING]
    ss = scr[b + 3 * RING:b + 4 * RING]
    cid = lax.axis_index("c")
    sid = lax.axis_index("s")

    def run(x, src, dst, out):
        r0 = sid * RPT
        pltpu.sync_copy(zeros.at[pl.ds(r0, RPT)], acc.at[pl.ds(r0, RPT)])
        plsc.subcore_barrier()
        base = sid * EPT

        def idx(arr, v, buf, sem):
            off = pl.multiple_of(base + v * CHUNK, CHUNK)
            pltpu.async_copy(arr.at[pl.ds(off, CHUNK)], buf, sem)

        LA = RING - 1
        for k in range(RING):
            idx(src, k, si[k], sr[k])
        for k in range(LA):
            idx(dst, k, di[k], sd[k])
        for k in range(LA):
            pltpu.make_async_copy(src.at[pl.ds(base, CHUNK)], si[k], sr[k]).wait()
            pltpu.async_copy(x.at[si[k]], rows[k], sg[k])

        def visit(v, k):
            bn = (k + LA) % RING
            pltpu.make_async_copy(x.at[si[k]], rows[k], sg[k]).wait()
            pltpu.make_async_copy(dst.at[pl.ds(base, CHUNK)], di[k], sd[k]).wait()
            pltpu.sync_copy(rows[k], acc.at[di[k]], add=True)

            @pl.when(v + RING < NCHUNK)
            def _():
                idx(src, v + RING, si[k], sr[k])

            @pl.when(v + LA < NCHUNK)
            def _():
                idx(dst, v + LA, di[bn], sd[bn])
                pltpu.make_async_copy(src.at[pl.ds(base, CHUNK)], si[bn], sr[bn]).wait()
                pltpu.async_copy(x.at[si[bn]], rows[bn], sg[bn])

        def block(j, carry):
            for k in range(RING):
                visit(RING * j + k, k)
            return carry

        lax.fori_loop(0, NCHUNK // RING, block, 0)
        plsc.subcore_barrier()
        pltpu.sync_copy(acc.at[pl.ds(r0, RPT)], out.at[pl.ds(r0, RPT)])

    @pl.when(cid == 0)
    def _():
        run(x0, src0, dst0, out0)

    @pl.when(cid == 1)
    def _():
        run(x1, src1, dst1, out1)


def _spmm_pair(x0, src0, dst0, x1, src1, dst1, zeros):
    f = pl.kernel(
        _spmm_body,
        out_type=(jax.ShapeDtypeStruct((N_PAD, D), jnp.float32),
                  jax.ShapeDtypeStruct((N_PAD, D), jnp.float32)),
        mesh=plsc.VectorSubcoreMesh(core_axis_name="c", subcore_axis_name="s"),
        scratch_types=(
            [pltpu.VMEM((CHUNK,), jnp.int32)] * (2 * RING)
            + [pltpu.VMEM((CHUNK, D), jnp.float32)] * RING
            + [pltpu.VMEM_SHARED((N_PAD, D), jnp.float32)]
            + [pltpu.SemaphoreType.DMA] * (4 * RING)
        ),
    )
    return f(x0, src0, dst0, x1, src1, dst1, zeros)


def _leaky(x):
    return jnp.where(x >= 0, x, ALPHA * x)


BT = 2000
BT_H = 1264


def _hidden_body(p0, p1, w0, w1, h0, h1):
    h0[...] = _leaky(jnp.dot(p0[...], w0[...], preferred_element_type=jnp.float32))
    h1[...] = _leaky(jnp.dot(p1[...], w1[...], preferred_element_type=jnp.float32))


def _hidden(p10, p11, W1_0, W1_1):
    row = pl.BlockSpec((BT_H, D), lambda i: (i, 0))
    wsp = pl.BlockSpec((D, D), lambda i: (0, 0))
    return pl.pallas_call(
        _hidden_body,
        grid=(N_PAD // BT_H,),
        in_specs=[row, row, wsp, wsp],
        out_specs=[row, row],
        out_shape=(jax.ShapeDtypeStruct((N_PAD, D), jnp.float32),
                   jax.ShapeDtypeStruct((N_PAD, D), jnp.float32)),
    )(p10, p11, W1_0, W1_1)


def _final_body(p20, uf0, p21, uf1,
                w2m0, w2s0, umt0, umb0, ust0, usb0, mb0, sb0,
                w2m1, w2s1, umt1, umb1, ust1, usb1, mb1, sb1,
                vm, vs):
    def dom(p2, uf, w2m, w2s, umt, umb, ust, usb, mb, sb):
        mh = _leaky(jnp.dot(p2[...], w2m[...], preferred_element_type=jnp.float32))
        lh = _leaky(jnp.dot(p2[...], w2s[...], preferred_element_type=jnp.float32))
        mean = (jnp.dot(mh, umt[...], preferred_element_type=jnp.float32)
                + jnp.dot(uf[...], umb[...], preferred_element_type=jnp.float32)
                + mb[...])
        logstd = (jnp.dot(lh, ust[...], preferred_element_type=jnp.float32)
                  + jnp.dot(uf[...], usb[...], preferred_element_type=jnp.float32)
                  + sb[...])
        return mean, logstd

    m0, s0 = dom(p20, uf0, w2m0, w2s0, umt0, umb0, ust0, usb0, mb0, sb0)
    m1, s1 = dom(p21, uf1, w2m1, w2s1, umt1, umb1, ust1, usb1, mb1, sb1)
    vm[...] = RATE * m0 + (1.0 - RATE) * m1
    vs[...] = RATE * s0 + (1.0 - RATE) * s1


def _final(p20, uf0, p21, uf1, weights):
    row = pl.BlockSpec((BT, D), lambda i: (i, 0))
    wsp = pl.BlockSpec((D, D), lambda i: (0, 0))
    bsp = pl.BlockSpec((1, D), lambda i: (0, 0))
    w_specs = ([wsp] * 6 + [bsp] * 2) * 2
    return pl.pallas_call(
        _final_body,
        grid=(N // BT,),
        in_specs=[row, row, row, row] + w_specs,
        out_specs=[row, row],
        out_shape=(jax.ShapeDtypeStruct((N, D), jnp.float32),
                   jax.ShapeDtypeStruct((N, D), jnp.float32)),
    )(p20, uf0, p21, uf1, *weights)


def kernel(ufea_0, ufea_1, edge_index_0, edge_index_1,
           W1_0, W2m_0, W2s_0, UmW_0, Umb_0, UsW_0, Usb_0,
           W1_1, W2m_1, W2s_1, UmW_1, Umb_1, UsW_1, Usb_1):
    npad = E_PAD - E
    pad_src = N + (jnp.arange(npad, dtype=jnp.int32) % (N_PAD - N))
    pad_dst = jnp.arange(npad, dtype=jnp.int32) % N
    pad_e = jnp.stack([pad_src, pad_dst])
    ei0 = jnp.concatenate([edge_index_0, pad_e], axis=1)
    ei1 = jnp.concatenate([edge_index_1, pad_e], axis=1)
    src0, dst0 = ei0[0], ei0[1]
    src1, dst1 = ei1[0], ei1[1]
    pad_x = jnp.zeros((N_PAD - N, D), jnp.float32)
    uf0p = jnp.concatenate([ufea_0, pad_x], axis=0)
    uf1p = jnp.concatenate([ufea_1, pad_x], axis=0)
    zeros = jnp.zeros((N_PAD, D), jnp.float32)

    p10, p11 = _spmm_pair(uf0p, src0, dst0, uf1p, src1, dst1, zeros)
    h0, h1 = _hidden(p10, p11, W1_0, W1_1)
    p20, p21 = _spmm_pair(h0, dst0, src0, h1, dst1, src1, zeros)

    weights = (
        W2m_0, W2s_0, UmW_0[:D], UmW_0[D:], UsW_0[:D], UsW_0[D:],
        Umb_0.reshape(1, D), Usb_0.reshape(1, D),
        W2m_1, W2s_1, UmW_1[:D], UmW_1[D:], UsW_1[:D], UsW_1[D:],
        Umb_1.reshape(1, D), Usb_1.reshape(1, D),
    )
    return _final(p20, ufea_0, p21, ufea_1, weights)


# speedup vs baseline: 1.9969x; 1.0076x over previous
import jax
import jax.numpy as jnp
from jax import lax
from jax.experimental import pallas as pl
from jax.experimental.pallas import tpu as pltpu
from jax.experimental.pallas import tpu_sc as plsc

N = 10000
E = 320000
D = 128
ALPHA = 0.2
RATE = 0.5

NS = 16
CHUNK = 64
RING = 5
NCHUNK = 315
EPT = NCHUNK * CHUNK
E_PAD = NS * EPT
RPT = 632
N_PAD = NS * RPT


def _spmm_body(x0, src0, dst0, x1, src1, dst1, zeros, out0, out1, *scr):
    si = scr[0:RING]
    di = scr[RING:2 * RING]
    rows = scr[2 * RING:3 * RING]
    acc = scr[3 * RING]
    b = 3 * RING + 1
    sr = scr[b:b + RING]
    sd = scr[b + RING:b + 2 * RING]
    sg = scr[b + 2 * RING:b + 3 * RING]
    ss = scr[b + 3 * RING:b + 4 * RING]
    cid = lax.axis_index("c")
    sid = lax.axis_index("s")

    def run(x, src, dst, out):
        r0 = sid * RPT
        pltpu.sync_copy(zeros.at[pl.ds(r0, RPT)], acc.at[pl.ds(r0, RPT)])
        plsc.subcore_barrier()
        base = sid * EPT

        def idx(arr, v, buf, sem):
            off = pl.multiple_of(base + v * CHUNK, CHUNK)
            pltpu.async_copy(arr.at[pl.ds(off, CHUNK)], buf, sem)

        LA = RING - 1
        for k in range(RING):
            idx(src, k, si[k], sr[k])
        for k in range(LA):
            idx(dst, k, di[k], sd[k])
        for k in range(LA):
            pltpu.make_async_copy(src.at[pl.ds(base, CHUNK)], si[k], sr[k]).wait()
            pltpu.async_copy(x.at[si[k]], rows[k], sg[k])

        def visit(v, k):
            bn = (k + LA) % RING
            pltpu.make_async_copy(x.at[si[k]], rows[k], sg[k]).wait()
            pltpu.make_async_copy(dst.at[pl.ds(base, CHUNK)], di[k], sd[k]).wait()
            pltpu.sync_copy(rows[k], acc.at[di[k]], add=True)

            @pl.when(v + RING < NCHUNK)
            def _():
                idx(src, v + RING, si[k], sr[k])

            @pl.when(v + LA < NCHUNK)
            def _():
                idx(dst, v + LA, di[bn], sd[bn])
                pltpu.make_async_copy(src.at[pl.ds(base, CHUNK)], si[bn], sr[bn]).wait()
                pltpu.async_copy(x.at[si[bn]], rows[bn], sg[bn])

        def block(j, carry):
            for k in range(RING):
                visit(RING * j + k, k)
            return carry

        lax.fori_loop(0, NCHUNK // RING, block, 0)
        plsc.subcore_barrier()
        pltpu.sync_copy(acc.at[pl.ds(r0, RPT)], out.at[pl.ds(r0, RPT)])

    @pl.when(cid == 0)
    def _():
        run(x0, src0, dst0, out0)

    @pl.when(cid == 1)
    def _():
        run(x1, src1, dst1, out1)


def _spmm_pair(x0, src0, dst0, x1, src1, dst1, zeros):
    f = pl.kernel(
        _spmm_body,
        out_type=(jax.ShapeDtypeStruct((N_PAD, D), jnp.float32),
                  jax.ShapeDtypeStruct((N_PAD, D), jnp.float32)),
        mesh=plsc.VectorSubcoreMesh(core_axis_name="c", subcore_axis_name="s"),
        scratch_types=(
            [pltpu.VMEM((CHUNK,), jnp.int32)] * (2 * RING)
            + [pltpu.VMEM((CHUNK, D), jnp.float32)] * RING
            + [pltpu.VMEM_SHARED((N_PAD, D), jnp.float32)]
            + [pltpu.SemaphoreType.DMA] * (4 * RING)
        ),
    )
    return f(x0, src0, dst0, x1, src1, dst1, zeros)


def _leaky(x):
    return jnp.where(x >= 0, x, ALPHA * x)


BT = 2000
BT_H = 1264


def _hidden_body(p0, p1, w0, w1, h0, h1):
    h0[...] = _leaky(jnp.dot(p0[...], w0[...], preferred_element_type=jnp.float32))
    h1[...] = _leaky(jnp.dot(p1[...], w1[...], preferred_element_type=jnp.float32))


def _hidden(p10, p11, W1_0, W1_1):
    row = pl.BlockSpec((BT_H, D), lambda i: (i, 0))
    wsp = pl.BlockSpec((D, D), lambda i: (0, 0))
    return pl.pallas_call(
        _hidden_body,
        grid=(N_PAD // BT_H,),
        in_specs=[row, row, wsp, wsp],
        out_specs=[row, row],
        out_shape=(jax.ShapeDtypeStruct((N_PAD, D), jnp.float32),
                   jax.ShapeDtypeStruct((N_PAD, D), jnp.float32)),
    )(p10, p11, W1_0, W1_1)


def _final_body(p20, uf0, p21, uf1,
                w2m0, w2s0, umt0, umb0, ust0, usb0, mb0, sb0,
                w2m1, w2s1, umt1, umb1, ust1, usb1, mb1, sb1,
                vm, vs):
    def dom(p2, uf, w2m, w2s, umt, umb, ust, usb, mb, sb):
        mh = _leaky(jnp.dot(p2[...], w2m[...], preferred_element_type=jnp.float32))
        lh = _leaky(jnp.dot(p2[...], w2s[...], preferred_element_type=jnp.float32))
        mean = (jnp.dot(mh, umt[...], preferred_element_type=jnp.float32)
                + jnp.dot(uf[...], umb[...], preferred_element_type=jnp.float32)
                + mb[...])
        logstd = (jnp.dot(lh, ust[...], preferred_element_type=jnp.float32)
                  + jnp.dot(uf[...], usb[...], preferred_element_type=jnp.float32)
                  + sb[...])
        return mean, logstd

    m0, s0 = dom(p20, uf0, w2m0, w2s0, umt0, umb0, ust0, usb0, mb0, sb0)
    m1, s1 = dom(p21, uf1, w2m1, w2s1, umt1, umb1, ust1, usb1, mb1, sb1)
    vm[...] = RATE * m0 + (1.0 - RATE) * m1
    vs[...] = RATE * s0 + (1.0 - RATE) * s1


def _final(p20, uf0, p21, uf1, weights):
    row = pl.BlockSpec((BT, D), lambda i: (i, 0))
    wsp = pl.BlockSpec((D, D), lambda i: (0, 0))
    bsp = pl.BlockSpec((1, D), lambda i: (0, 0))
    w_specs = ([wsp] * 6 + [bsp] * 2) * 2
    return pl.pallas_call(
        _final_body,
        grid=(N // BT,),
        in_specs=[row, row, row, row] + w_specs,
        out_specs=[row, row],
        out_shape=(jax.ShapeDtypeStruct((N, D), jnp.float32),
                   jax.ShapeDtypeStruct((N, D), jnp.float32)),
    )(p20, uf0, p21, uf1, *weights)


def kernel(ufea_0, ufea_1, edge_index_0, edge_index_1,
           W1_0, W2m_0, W2s_0, UmW_0, Umb_0, UsW_0, Usb_0,
           W1_1, W2m_1, W2s_1, UmW_1, Umb_1, UsW_1, Usb_1):
    npad = E_PAD - E
    pad_src = N + (jnp.arange(npad, dtype=jnp.int32) % (N_PAD - N))
    pad_dst = jnp.arange(npad, dtype=jnp.int32) % N
    pad_e = jnp.stack([pad_src, pad_dst])
    ei0 = jnp.concatenate([edge_index_0, pad_e], axis=1)
    ei1 = jnp.concatenate([edge_index_1, pad_e], axis=1)
    src0, dst0 = ei0[0], ei0[1]
    src1, dst1 = ei1[0], ei1[1]
    pad_x = jnp.zeros((N_PAD - N, D), jnp.float32)
    uf0p = jnp.concatenate([ufea_0, pad_x], axis=0)
    uf1p = jnp.concatenate([ufea_1, pad_x], axis=0)
    zeros = jnp.zeros((N_PAD, D), jnp.float32)

    p10, p11 = _spmm_pair(uf0p, src0, dst0, uf1p, src1, dst1, zeros)
    h0, h1 = _hidden(p10, p11, W1_0, W1_1)
    p20, p21 = _spmm_pair(h0, dst0, src0, h1, dst1, src1, zeros)

    weights = (
        W2m_0, W2s_0, UmW_0[:D], UmW_0[D:], UsW_0[:D], UsW_0[D:],
        Umb_0.reshape(1, D), Usb_0.reshape(1, D),
        W2m_1, W2s_1, UmW_1[:D], UmW_1[D:], UsW_1[:D], UsW_1[D:],
        Umb_1.reshape(1, D), Usb_1.reshape(1, D),
    )
    return _final(p20, ufea_0, p21, ufea_1, weights)
